# Initial kernel scaffold; baseline (speedup 1.0000x reference)
import jax
import jax.numpy as jnp
from jax.experimental import pallas as pl


def kernel(indices, table):
    raise NotImplementedError("write your pallas kernel here")


# speedup vs baseline: 1.4970x; 1.4970x over previous
import functools

import jax
import jax.numpy as jnp
from jax import lax
from jax.experimental import pallas as pl
from jax.experimental.pallas import tpu as pltpu
from jax.experimental.pallas import tpu_sc as plsc

NUM_EMBEDDINGS = 1000000
EMBED_DIM = 32
BATCH = 4096
SEQ = 200

N = BATCH * SEQ
NC, NS = 2, 16
NW = NC * NS
PER_W = N // NW
CHUNK = 3200
N_CHUNKS = PER_W // CHUNK


@functools.partial(
    pl.kernel,
    out_type=jax.ShapeDtypeStruct((N, EMBED_DIM), jnp.float32),
    mesh=plsc.VectorSubcoreMesh(core_axis_name="c", subcore_axis_name="s"),
    compiler_params=pltpu.CompilerParams(use_tc_tiling_on_sc=False),
    scratch_types=[
        pltpu.VMEM((CHUNK,), jnp.int32),
        pltpu.VMEM((CHUNK, EMBED_DIM), jnp.float32),
        pltpu.SemaphoreType.DMA,
    ],
)
def _gather_kernel(idx_hbm, table_hbm, out_hbm, idx_v, rows_v, sem):
    wid = lax.axis_index("s") * NC + lax.axis_index("c")
    base = wid * PER_W

    def body(i, carry):
        off = base + i * CHUNK
        pltpu.sync_copy(idx_hbm.at[pl.ds(off, CHUNK)], idx_v)
        pltpu.async_copy(table_hbm.at[idx_v], rows_v, sem).wait()
        pltpu.sync_copy(rows_v, out_hbm.at[pl.ds(off, CHUNK)])
        return carry

    lax.fori_loop(0, N_CHUNKS, body, 0)


def kernel(indices, table):
    idx_flat = indices.reshape(N)
    out = _gather_kernel(idx_flat, table)
    return out.reshape(BATCH, SEQ, EMBED_DIM)


# speedup vs baseline: 1.5005x; 1.0024x over previous
import functools

import jax
import jax.numpy as jnp
from jax import lax
from jax.experimental import pallas as pl
from jax.experimental.pallas import tpu as pltpu
from jax.experimental.pallas import tpu_sc as plsc

NUM_EMBEDDINGS = 1000000
EMBED_DIM = 32
BATCH = 4096
SEQ = 200

N = BATCH * SEQ
NC, NS = 2, 16
NW = NC * NS
PER_W = N // NW
CHUNK = 1600
N_CHUNKS = PER_W // CHUNK


@functools.partial(
    pl.kernel,
    out_type=jax.ShapeDtypeStruct((N, EMBED_DIM), jnp.float32),
    mesh=plsc.VectorSubcoreMesh(core_axis_name="c", subcore_axis_name="s"),
    compiler_params=pltpu.CompilerParams(use_tc_tiling_on_sc=False),
    scratch_types=[
        pltpu.VMEM((PER_W,), jnp.int32),
        pltpu.VMEM((CHUNK, EMBED_DIM), jnp.float32),
        pltpu.VMEM((CHUNK, EMBED_DIM), jnp.float32),
        pltpu.SemaphoreType.DMA,
        pltpu.SemaphoreType.DMA,
        pltpu.SemaphoreType.DMA,
        pltpu.SemaphoreType.DMA,
    ],
)
def _gather_kernel(idx_hbm, table_hbm, out_hbm, idx_v, rows0, rows1,
                   sg0, sg1, so0, so1):
    wid = lax.axis_index("s") * NC + lax.axis_index("c")
    base = wid * PER_W
    rows = (rows0, rows1)
    sg = (sg0, sg1)
    so = (so0, so1)

    pltpu.sync_copy(idx_hbm.at[pl.ds(base, PER_W)], idx_v)

    def gather(i, b):
        return pltpu.async_copy(
            table_hbm.at[idx_v.at[pl.ds(i * CHUNK, CHUNK)]], rows[b], sg[b])

    def store(i, b):
        return pltpu.async_copy(
            rows[b], out_hbm.at[pl.ds(base + i * CHUNK, CHUNK)], so[b])

    g_desc = [None, None]
    o_desc = [None, None]
    g_desc[0] = gather(0, 0)
    for i in range(N_CHUNKS):
        b = i % 2
        nb = 1 - b
        if i + 1 < N_CHUNKS:
            if o_desc[nb] is not None:
                o_desc[nb].wait()
            g_desc[nb] = gather(i + 1, nb)
        g_desc[b].wait()
        o_desc[b] = store(i, b)
    o_desc[0].wait()
    o_desc[1].wait()


def kernel(indices, table):
    idx_flat = indices.reshape(N)
    out = _gather_kernel(idx_flat, table)
    return out.reshape(BATCH, SEQ, EMBED_DIM)
